# baseline (speedup 1.0000x reference)
import functools

import jax
import jax.numpy as jnp
from jax import lax
from jax.experimental import pallas as pl
from jax.experimental.pallas import tpu as pltpu
from jax.experimental.pallas import tpu_sc as plsc

_NUM_CORES = 2
_NUM_SUBCORES = 16
_NUM_WORKERS = _NUM_CORES * _NUM_SUBCORES
_LANES = 16


def _fuse_weights_kernel(table_ref, we_ref, wd_ref, bd_ref, out_ref, *, v, h, r, kh):
    table = table_ref[...]
    wd = wd_ref[...]
    we = we_ref[...]
    bd = bd_ref[...]
    dn = (((1,), (1,)), ((), ()))
    ta_t = lax.dot_general(wd[:, 0:h], table, dn,
                           preferred_element_type=jnp.float32)
    tb_t = lax.dot_general(wd[:, h:2 * h], table, dn,
                           preferred_element_type=jnp.float32)
    d_t = lax.dot_general(wd[:, 2 * h:2 * h + r], we, (((1,), (0,)), ((), ())),
                          preferred_element_type=jnp.float32)
    ones = jnp.ones((1, kh), jnp.float32)
    bd_col = lax.dot_general(bd, ones, (((0,), (0,)), ((), ())),
                             preferred_element_type=jnp.float32)
    ta_t = ta_t + bd_col[:, 0:v]
    out_ref[...] = jnp.zeros(out_ref.shape, jnp.bfloat16)
    out_ref[:, 0:v] = ta_t.astype(jnp.bfloat16)
    out_ref[:, kh:kh + v] = tb_t.astype(jnp.bfloat16)
    out_ref[:, 2 * kh:2 * kh + r] = d_t.astype(jnp.bfloat16)


def _zgather_kernel(nbr_ref, z_ref, zs_ref, zd_ref, nbr_v, out_v, z_v, *,
                    half, n_edges):
    wid = lax.axis_index("s") * _NUM_CORES + lax.axis_index("c")
    base = wid * half
    pltpu.sync_copy(z_ref, z_v)

    for p, oref in enumerate((zs_ref, zd_ref)):
        pltpu.sync_copy(nbr_ref.at[pl.ds(p * n_edges + base, half)], nbr_v)

        def body(i, carry):
            idx = nbr_v[pl.ds(i * _LANES, _LANES)]
            out_v[pl.ds(i * _LANES, _LANES)] = plsc.load_gather(z_v, [idx])
            return carry

        lax.fori_loop(0, half // _LANES, body, 0)
        pltpu.sync_copy(out_v, oref.at[pl.ds(base, half)])


def _edge_kernel(zs_ref, zd_ref, e_ref, w_ref, out_ref, *, b, kh):
    zs = zs_ref[...].reshape(1, b)
    zd = zd_ref[...].reshape(1, b)
    iota = lax.broadcasted_iota(jnp.int32, (kh, b), 0)
    ohs = (iota == zs).astype(jnp.bfloat16)
    ohd = (iota == zd).astype(jnp.bfloat16)
    et = e_ref[...].astype(jnp.bfloat16)
    x = jnp.concatenate([ohs, ohd, et], axis=0)
    lin = lax.dot_general(w_ref[...], x, (((1,), (0,)), ((), ())),
                          preferred_element_type=jnp.float32)
    out_ref[...] = (0.5 * lin) * (1.0 + jnp.tanh(0.5 * lin))


def kernel(e_rbf, z, nbr_list, W_edge, table, Wd, bd):
    n_edges, r = e_rbf.shape
    v, h = table.shape
    c = Wd.shape[0]
    kh = 128
    k = 2 * kh + r

    z = z.astype(jnp.int32)
    nbr_flat = nbr_list.astype(jnp.int32).T.reshape(-1)
    half = n_edges // _NUM_WORKERS

    zgather = pl.kernel(
        functools.partial(_zgather_kernel, half=half, n_edges=n_edges),
        out_type=(jax.ShapeDtypeStruct((n_edges,), jnp.int32),
                  jax.ShapeDtypeStruct((n_edges,), jnp.int32)),
        mesh=plsc.VectorSubcoreMesh(
            core_axis_name="c", subcore_axis_name="s",
            num_cores=_NUM_CORES, num_subcores=_NUM_SUBCORES),
        compiler_params=pltpu.CompilerParams(needs_layout_passes=False),
        scratch_types=[
            pltpu.VMEM((half,), jnp.int32),
            pltpu.VMEM((half,), jnp.int32),
            pltpu.VMEM((z.shape[0],), jnp.int32),
        ],
    )
    zs_flat, zd_flat = zgather(nbr_flat, z)

    wtilde_t = pl.pallas_call(
        functools.partial(_fuse_weights_kernel, v=v, h=h, r=r, kh=kh),
        out_shape=jax.ShapeDtypeStruct((c, k), jnp.bfloat16),
    )(table, W_edge, Wd, bd.reshape(1, c))

    b = 3200
    grid = n_edges // b
    zs_v = zs_flat.reshape(grid, 1, b)
    zd_v = zd_flat.reshape(grid, 1, b)
    e_t = e_rbf.T

    out_t = pl.pallas_call(
        functools.partial(_edge_kernel, b=b, kh=kh),
        grid=(grid,),
        in_specs=[
            pl.BlockSpec((1, 1, b), lambda i: (i, 0, 0)),
            pl.BlockSpec((1, 1, b), lambda i: (i, 0, 0)),
            pl.BlockSpec((r, b), lambda i: (0, i)),
            pl.BlockSpec((c, k), lambda i: (0, 0)),
        ],
        out_specs=pl.BlockSpec((c, b), lambda i: (0, i)),
        out_shape=jax.ShapeDtypeStruct((c, n_edges), jnp.float32),
    )(zs_v, zd_v, e_t, wtilde_t)
    return out_t.T


# speedup vs baseline: 1.2406x; 1.2406x over previous
import functools

import jax
import jax.numpy as jnp
from jax import lax
from jax.experimental import pallas as pl
from jax.experimental.pallas import tpu as pltpu
from jax.experimental.pallas import tpu_sc as plsc

_NUM_CORES = 2
_NUM_SUBCORES = 16
_NUM_WORKERS = _NUM_CORES * _NUM_SUBCORES
_LANES = 16


def _fuse_weights_kernel(table_ref, we_ref, wd_ref, bd_ref, out_ref, *, v, h, r, kh):
    table = table_ref[...]
    wd = wd_ref[...]
    we = we_ref[...]
    bd = bd_ref[...]
    dn = (((1,), (1,)), ((), ()))
    ta_t = lax.dot_general(wd[:, 0:h], table, dn,
                           preferred_element_type=jnp.float32)
    tb_t = lax.dot_general(wd[:, h:2 * h], table, dn,
                           preferred_element_type=jnp.float32)
    d_t = lax.dot_general(wd[:, 2 * h:2 * h + r], we, (((1,), (0,)), ((), ())),
                          preferred_element_type=jnp.float32)
    ones = jnp.ones((1, kh), jnp.float32)
    bd_col = lax.dot_general(bd, ones, (((0,), (0,)), ((), ())),
                             preferred_element_type=jnp.float32)
    ta_t = ta_t + bd_col[:, 0:v]
    out_ref[...] = jnp.zeros(out_ref.shape, jnp.bfloat16)
    out_ref[:, 0:v] = ta_t.astype(jnp.bfloat16)
    out_ref[:, kh:kh + v] = tb_t.astype(jnp.bfloat16)
    out_ref[:, 2 * kh:2 * kh + r] = d_t.astype(jnp.bfloat16)


def _zgather_kernel(nbr_ref, z_ref, zs_ref, zd_ref, nbr_v, out_v, z_v, *,
                    half, n_edges):
    wid = lax.axis_index("s") * _NUM_CORES + lax.axis_index("c")
    base = wid * half
    pltpu.sync_copy(z_ref, z_v)

    for p, oref in enumerate((zs_ref, zd_ref)):
        pltpu.sync_copy(nbr_ref.at[pl.ds(p * n_edges + base, half)], nbr_v)

        def body(i, carry):
            idx = nbr_v[pl.ds(i * _LANES, _LANES)]
            out_v[pl.ds(i * _LANES, _LANES)] = plsc.load_gather(z_v, [idx])
            return carry

        lax.fori_loop(0, half // _LANES, body, 0)
        pltpu.sync_copy(out_v, oref.at[pl.ds(base, half)])


def _edge_kernel(zs_ref, zd_ref, e_ref, w_ref, out_ref, *, b, kh):
    zs = zs_ref[...].reshape(1, b)
    zd = zd_ref[...].reshape(1, b)
    iota = lax.broadcasted_iota(jnp.int32, (kh, b), 0)
    ohs = (iota == zs).astype(jnp.bfloat16)
    ohd = (iota == zd).astype(jnp.bfloat16)
    et = e_ref[...].astype(jnp.bfloat16)
    x = jnp.concatenate([ohs, ohd, et], axis=0)
    lin = lax.dot_general(w_ref[...], x, (((1,), (0,)), ((), ())),
                          preferred_element_type=jnp.float32)
    h = 0.5 * lin
    out_ref[...] = h * jnp.tanh(h) + h


def kernel(e_rbf, z, nbr_list, W_edge, table, Wd, bd):
    n_edges, r = e_rbf.shape
    v, h = table.shape
    c = Wd.shape[0]
    kh = 104
    k = 2 * kh + r

    z = z.astype(jnp.int32)
    nbr_flat = nbr_list.astype(jnp.int32).T.reshape(-1)
    half = n_edges // _NUM_WORKERS

    zgather = pl.kernel(
        functools.partial(_zgather_kernel, half=half, n_edges=n_edges),
        out_type=(jax.ShapeDtypeStruct((n_edges,), jnp.int32),
                  jax.ShapeDtypeStruct((n_edges,), jnp.int32)),
        mesh=plsc.VectorSubcoreMesh(
            core_axis_name="c", subcore_axis_name="s",
            num_cores=_NUM_CORES, num_subcores=_NUM_SUBCORES),
        compiler_params=pltpu.CompilerParams(needs_layout_passes=False),
        scratch_types=[
            pltpu.VMEM((half,), jnp.int32),
            pltpu.VMEM((half,), jnp.int32),
            pltpu.VMEM((z.shape[0],), jnp.int32),
        ],
    )
    zs_flat, zd_flat = zgather(nbr_flat, z)

    wtilde_t = pl.pallas_call(
        functools.partial(_fuse_weights_kernel, v=v, h=h, r=r, kh=kh),
        out_shape=jax.ShapeDtypeStruct((c, k), jnp.bfloat16),
    )(table, W_edge, Wd, bd.reshape(1, c))

    b = 6400
    grid = n_edges // b
    zs_v = zs_flat.reshape(grid, 1, b)
    zd_v = zd_flat.reshape(grid, 1, b)
    e_t = e_rbf.T

    out_t = pl.pallas_call(
        functools.partial(_edge_kernel, b=b, kh=kh),
        grid=(grid,),
        in_specs=[
            pl.BlockSpec((1, 1, b), lambda i: (i, 0, 0)),
            pl.BlockSpec((1, 1, b), lambda i: (i, 0, 0)),
            pl.BlockSpec((r, b), lambda i: (0, i)),
            pl.BlockSpec((c, k), lambda i: (0, 0)),
        ],
        out_specs=pl.BlockSpec((c, b), lambda i: (0, i)),
        out_shape=jax.ShapeDtypeStruct((c, n_edges), jnp.float32),
    )(zs_v, zd_v, e_t, wtilde_t)
    return out_t.T


# speedup vs baseline: 1.2939x; 1.0430x over previous
import functools

import jax
import jax.numpy as jnp
from jax import lax
from jax.experimental import pallas as pl
from jax.experimental.pallas import tpu as pltpu
from jax.experimental.pallas import tpu_sc as plsc

_NUM_CORES = 2
_NUM_SUBCORES = 16
_NUM_WORKERS = _NUM_CORES * _NUM_SUBCORES
_LANES = 16


def _fuse_weights_kernel(table_ref, we_ref, wd_ref, bd_ref, out_ref, *, v, h, r, kh):
    table = table_ref[...]
    wd = wd_ref[...]
    we = we_ref[...]
    bd = bd_ref[...]
    dn = (((1,), (1,)), ((), ()))
    ta_t = lax.dot_general(wd[:, 0:h], table, dn,
                           preferred_element_type=jnp.float32)
    tb_t = lax.dot_general(wd[:, h:2 * h], table, dn,
                           preferred_element_type=jnp.float32)
    d_t = lax.dot_general(wd[:, 2 * h:2 * h + r], we, (((1,), (0,)), ((), ())),
                          preferred_element_type=jnp.float32)
    ones = jnp.ones((1, kh), jnp.float32)
    bd_col = lax.dot_general(bd, ones, (((0,), (0,)), ((), ())),
                             preferred_element_type=jnp.float32)
    ta_t = ta_t + bd_col[:, 0:v]
    out_ref[...] = jnp.zeros(out_ref.shape, jnp.bfloat16)
    out_ref[:, 0:v] = ta_t.astype(jnp.bfloat16)
    out_ref[:, kh:kh + v] = tb_t.astype(jnp.bfloat16)
    out_ref[:, 2 * kh:2 * kh + r] = d_t.astype(jnp.bfloat16)


def _zgather_kernel(nbr_ref, z_ref, zs_ref, zd_ref, nbr_v, out_v, z_v, *,
                    half, n_edges):
    wid = lax.axis_index("s") * _NUM_CORES + lax.axis_index("c")
    base = wid * half
    pltpu.sync_copy(z_ref, z_v)

    for p, oref in enumerate((zs_ref, zd_ref)):
        pltpu.sync_copy(nbr_ref.at[pl.ds(p * n_edges + base, half)], nbr_v)

        def body(i, carry):
            for u in range(5):
                j = i * 5 + u
                idx = nbr_v[pl.ds(j * _LANES, _LANES)]
                out_v[pl.ds(j * _LANES, _LANES)] = plsc.load_gather(z_v, [idx])
            return carry

        lax.fori_loop(0, half // (5 * _LANES), body, 0)
        pltpu.sync_copy(out_v, oref.at[pl.ds(base, half)])


def _edge_kernel(zs_ref, zd_ref, e_ref, w_ref, out_ref, *, b, kh):
    zs = zs_ref[...].reshape(1, b)
    zd = zd_ref[...].reshape(1, b)
    iota = lax.broadcasted_iota(jnp.int32, (kh, b), 0)
    ohs = (iota == zs).astype(jnp.bfloat16)
    ohd = (iota == zd).astype(jnp.bfloat16)
    et = e_ref[...].astype(jnp.bfloat16)
    x = jnp.concatenate([ohs, ohd, et], axis=0)
    lin = lax.dot_general(w_ref[...], x, (((1,), (0,)), ((), ())),
                          preferred_element_type=jnp.float32)
    h = 0.5 * lin
    out_ref[...] = h * jnp.tanh(h) + h


def kernel(e_rbf, z, nbr_list, W_edge, table, Wd, bd):
    n_edges, r = e_rbf.shape
    v, h = table.shape
    c = Wd.shape[0]
    kh = 104
    k = 2 * kh + r

    z = z.astype(jnp.int32)
    nbr_flat = nbr_list.astype(jnp.int32).T.reshape(-1)
    half = n_edges // _NUM_WORKERS

    zgather = pl.kernel(
        functools.partial(_zgather_kernel, half=half, n_edges=n_edges),
        out_type=(jax.ShapeDtypeStruct((n_edges,), jnp.int32),
                  jax.ShapeDtypeStruct((n_edges,), jnp.int32)),
        mesh=plsc.VectorSubcoreMesh(
            core_axis_name="c", subcore_axis_name="s",
            num_cores=_NUM_CORES, num_subcores=_NUM_SUBCORES),
        compiler_params=pltpu.CompilerParams(needs_layout_passes=False),
        scratch_types=[
            pltpu.VMEM((half,), jnp.int32),
            pltpu.VMEM((half,), jnp.int32),
            pltpu.VMEM((z.shape[0],), jnp.int32),
        ],
    )
    zs_flat, zd_flat = zgather(nbr_flat, z)

    wtilde_t = pl.pallas_call(
        functools.partial(_fuse_weights_kernel, v=v, h=h, r=r, kh=kh),
        out_shape=jax.ShapeDtypeStruct((c, k), jnp.bfloat16),
    )(table, W_edge, Wd, bd.reshape(1, c))

    b = 12800
    grid = n_edges // b
    zs_v = zs_flat.reshape(grid, 1, b)
    zd_v = zd_flat.reshape(grid, 1, b)
    e_t = e_rbf.T

    out_t = pl.pallas_call(
        functools.partial(_edge_kernel, b=b, kh=kh),
        grid=(grid,),
        in_specs=[
            pl.BlockSpec((1, 1, b), lambda i: (i, 0, 0)),
            pl.BlockSpec((1, 1, b), lambda i: (i, 0, 0)),
            pl.BlockSpec((r, b), lambda i: (0, i)),
            pl.BlockSpec((c, k), lambda i: (0, 0)),
        ],
        out_specs=pl.BlockSpec((c, b), lambda i: (0, i)),
        out_shape=jax.ShapeDtypeStruct((c, n_edges), jnp.float32),
    )(zs_v, zd_v, e_t, wtilde_t)
    return out_t.T


# speedup vs baseline: 1.3098x; 1.0123x over previous
import functools

import jax
import jax.numpy as jnp
from jax import lax
from jax.experimental import pallas as pl
from jax.experimental.pallas import tpu as pltpu
from jax.experimental.pallas import tpu_sc as plsc

_NUM_CORES = 2
_NUM_SUBCORES = 16
_NUM_WORKERS = _NUM_CORES * _NUM_SUBCORES
_LANES = 16


def _fuse_weights_kernel(table_ref, we_ref, wd_ref, bd_ref, out_ref, *, v, h, r, kh):
    table = table_ref[...]
    wd = wd_ref[...]
    we = we_ref[...]
    bd = bd_ref[...]
    dn = (((1,), (1,)), ((), ()))
    ta_t = lax.dot_general(wd[:, 0:h], table, dn,
                           preferred_element_type=jnp.float32)
    tb_t = lax.dot_general(wd[:, h:2 * h], table, dn,
                           preferred_element_type=jnp.float32)
    d_t = lax.dot_general(wd[:, 2 * h:2 * h + r], we, (((1,), (0,)), ((), ())),
                          preferred_element_type=jnp.float32)
    ones = jnp.ones((1, kh), jnp.float32)
    bd_col = lax.dot_general(bd, ones, (((0,), (0,)), ((), ())),
                             preferred_element_type=jnp.float32)
    ta_t = ta_t + bd_col[:, 0:v]
    out_ref[...] = jnp.zeros(out_ref.shape, jnp.bfloat16)
    out_ref[:, 0:v] = ta_t.astype(jnp.bfloat16)
    out_ref[:, kh:kh + v] = tb_t.astype(jnp.bfloat16)
    out_ref[:, 2 * kh:2 * kh + r] = d_t.astype(jnp.bfloat16)


def _zgather_kernel(nbr_ref, z_ref, zs_ref, zd_ref, nbr_v, out_v, z_v, *,
                    half, n_edges):
    wid = lax.axis_index("s") * _NUM_CORES + lax.axis_index("c")
    base = wid * half
    pltpu.sync_copy(z_ref, z_v)

    for p, oref in enumerate((zs_ref, zd_ref)):
        pltpu.sync_copy(nbr_ref.at[pl.ds(p * n_edges + base, half)], nbr_v)

        def body(i, carry):
            for u in range(5):
                j = i * 5 + u
                idx = nbr_v[pl.ds(j * _LANES, _LANES)]
                out_v[pl.ds(j * _LANES, _LANES)] = plsc.load_gather(z_v, [idx])
            return carry

        lax.fori_loop(0, half // (5 * _LANES), body, 0)
        pltpu.sync_copy(out_v, oref.at[pl.ds(base, half)])


def _edge_kernel(zs_ref, zd_ref, e_ref, w_ref, out_ref, *, b, kh):
    zs = zs_ref[...].reshape(1, b)
    zd = zd_ref[...].reshape(1, b)
    iota = lax.broadcasted_iota(jnp.int32, (kh, b), 0)
    ohs = (iota == zs).astype(jnp.bfloat16)
    ohd = (iota == zd).astype(jnp.bfloat16)
    et = e_ref[...].astype(jnp.bfloat16)
    x = jnp.concatenate([ohs, ohd, et], axis=0)
    lin = lax.dot_general(w_ref[...], x, (((1,), (0,)), ((), ())),
                          preferred_element_type=jnp.float32)
    h = 0.5 * lin
    out_ref[...] = h * jnp.tanh(h) + h


def kernel(e_rbf, z, nbr_list, W_edge, table, Wd, bd):
    n_edges, r = e_rbf.shape
    v, h = table.shape
    c = Wd.shape[0]
    kh = 104
    k = 2 * kh + r

    z = z.astype(jnp.int32)
    nbr_flat = nbr_list.astype(jnp.int32).T.reshape(-1)
    half = n_edges // _NUM_WORKERS

    zgather = pl.kernel(
        functools.partial(_zgather_kernel, half=half, n_edges=n_edges),
        out_type=(jax.ShapeDtypeStruct((n_edges,), jnp.int32),
                  jax.ShapeDtypeStruct((n_edges,), jnp.int32)),
        mesh=plsc.VectorSubcoreMesh(
            core_axis_name="c", subcore_axis_name="s",
            num_cores=_NUM_CORES, num_subcores=_NUM_SUBCORES),
        compiler_params=pltpu.CompilerParams(needs_layout_passes=False),
        scratch_types=[
            pltpu.VMEM((half,), jnp.int32),
            pltpu.VMEM((half,), jnp.int32),
            pltpu.VMEM((z.shape[0],), jnp.int32),
        ],
    )
    zs_flat, zd_flat = zgather(nbr_flat, z)

    wtilde_t = pl.pallas_call(
        functools.partial(_fuse_weights_kernel, v=v, h=h, r=r, kh=kh),
        out_shape=jax.ShapeDtypeStruct((c, k), jnp.bfloat16),
    )(table, W_edge, Wd, bd.reshape(1, c))

    b = 16000
    grid = n_edges // b
    zs_v = zs_flat.reshape(grid, 1, b)
    zd_v = zd_flat.reshape(grid, 1, b)
    e_t = e_rbf.T

    out_t = pl.pallas_call(
        functools.partial(_edge_kernel, b=b, kh=kh),
        grid=(grid,),
        in_specs=[
            pl.BlockSpec((1, 1, b), lambda i: (i, 0, 0)),
            pl.BlockSpec((1, 1, b), lambda i: (i, 0, 0)),
            pl.BlockSpec((r, b), lambda i: (0, i)),
            pl.BlockSpec((c, k), lambda i: (0, 0)),
        ],
        out_specs=pl.BlockSpec((c, b), lambda i: (0, i)),
        out_shape=jax.ShapeDtypeStruct((c, n_edges), jnp.float32),
    )(zs_v, zd_v, e_t, wtilde_t)
    return out_t.T


# speedup vs baseline: 1.3237x; 1.0106x over previous
import functools

import jax
import jax.numpy as jnp
from jax import lax
from jax.experimental import pallas as pl
from jax.experimental.pallas import tpu as pltpu
from jax.experimental.pallas import tpu_sc as plsc

_NUM_CORES = 2
_NUM_SUBCORES = 16
_NUM_WORKERS = _NUM_CORES * _NUM_SUBCORES
_LANES = 16


def _fuse_weights_kernel(table_ref, we_ref, wd_ref, bd_ref, out_ref, *, v, h, r, kh):
    table = table_ref[...]
    wd = wd_ref[...]
    we = we_ref[...]
    bd = bd_ref[...]
    dn = (((1,), (1,)), ((), ()))
    ta_t = lax.dot_general(wd[:, 0:h], table, dn,
                           preferred_element_type=jnp.float32)
    tb_t = lax.dot_general(wd[:, h:2 * h], table, dn,
                           preferred_element_type=jnp.float32)
    d_t = lax.dot_general(wd[:, 2 * h:2 * h + r], we, (((1,), (0,)), ((), ())),
                          preferred_element_type=jnp.float32)
    ones = jnp.ones((1, kh), jnp.float32)
    bd_col = lax.dot_general(bd, ones, (((0,), (0,)), ((), ())),
                             preferred_element_type=jnp.float32)
    ta_t = ta_t + bd_col[:, 0:v]
    out_ref[...] = jnp.zeros(out_ref.shape, jnp.bfloat16)
    out_ref[:, 0:v] = ta_t.astype(jnp.bfloat16)
    out_ref[:, kh:kh + v] = tb_t.astype(jnp.bfloat16)
    out_ref[:, 2 * kh:2 * kh + r] = d_t.astype(jnp.bfloat16)


def _zgather_kernel(nbr_ref, z_ref, zs_ref, zd_ref,
                    ns_v, nd_v, os_v, od_v, z_v, sem_s, sem_d, sem_o, *,
                    half, n_edges):
    wid = lax.axis_index("s") * _NUM_CORES + lax.axis_index("c")
    base = wid * half
    cs = pltpu.async_copy(nbr_ref.at[pl.ds(base, half)], ns_v, sem_s)
    cd = pltpu.async_copy(nbr_ref.at[pl.ds(n_edges + base, half)], nd_v, sem_d)
    pltpu.sync_copy(z_ref, z_v)

    def gather(nbr_v, out_v):
        def body(i, carry):
            for u in range(5):
                j = i * 5 + u
                idx = nbr_v[pl.ds(j * _LANES, _LANES)]
                out_v[pl.ds(j * _LANES, _LANES)] = plsc.load_gather(z_v, [idx])
            return carry
        lax.fori_loop(0, half // (5 * _LANES), body, 0)

    cs.wait()
    gather(ns_v, os_v)
    co = pltpu.async_copy(os_v, zs_ref.at[pl.ds(base, half)], sem_o)
    cd.wait()
    gather(nd_v, od_v)
    co.wait()
    pltpu.sync_copy(od_v, zd_ref.at[pl.ds(base, half)])


def _edge_kernel(zs_ref, zd_ref, e_ref, w_ref, out_ref, *, b, kh):
    zs = zs_ref[...].reshape(1, b)
    zd = zd_ref[...].reshape(1, b)
    iota = lax.broadcasted_iota(jnp.int32, (kh, b), 0)
    ohs = (iota == zs).astype(jnp.bfloat16)
    ohd = (iota == zd).astype(jnp.bfloat16)
    et = e_ref[...].astype(jnp.bfloat16)
    x = jnp.concatenate([ohs, ohd, et], axis=0)
    lin = lax.dot_general(w_ref[...], x, (((1,), (0,)), ((), ())),
                          preferred_element_type=jnp.float32)
    h = 0.5 * lin
    out_ref[...] = h * jnp.tanh(h) + h


def kernel(e_rbf, z, nbr_list, W_edge, table, Wd, bd):
    n_edges, r = e_rbf.shape
    v, h = table.shape
    c = Wd.shape[0]
    kh = 104
    k = 2 * kh + r

    z = z.astype(jnp.int32)
    nbr_flat = nbr_list.astype(jnp.int32).T.reshape(-1)
    half = n_edges // _NUM_WORKERS

    zgather = pl.kernel(
        functools.partial(_zgather_kernel, half=half, n_edges=n_edges),
        out_type=(jax.ShapeDtypeStruct((n_edges,), jnp.int32),
                  jax.ShapeDtypeStruct((n_edges,), jnp.int32)),
        mesh=plsc.VectorSubcoreMesh(
            core_axis_name="c", subcore_axis_name="s",
            num_cores=_NUM_CORES, num_subcores=_NUM_SUBCORES),
        compiler_params=pltpu.CompilerParams(needs_layout_passes=False),
        scratch_types=[
            pltpu.VMEM((half,), jnp.int32),
            pltpu.VMEM((half,), jnp.int32),
            pltpu.VMEM((half,), jnp.int32),
            pltpu.VMEM((half,), jnp.int32),
            pltpu.VMEM((z.shape[0],), jnp.int32),
            pltpu.SemaphoreType.DMA,
            pltpu.SemaphoreType.DMA,
            pltpu.SemaphoreType.DMA,
        ],
    )
    zs_flat, zd_flat = zgather(nbr_flat, z)

    wtilde_t = pl.pallas_call(
        functools.partial(_fuse_weights_kernel, v=v, h=h, r=r, kh=kh),
        out_shape=jax.ShapeDtypeStruct((c, k), jnp.bfloat16),
    )(table, W_edge, Wd, bd.reshape(1, c))

    b = 16000
    grid = n_edges // b
    zs_v = zs_flat.reshape(grid, 1, b)
    zd_v = zd_flat.reshape(grid, 1, b)
    e_t = e_rbf.T

    out_t = pl.pallas_call(
        functools.partial(_edge_kernel, b=b, kh=kh),
        grid=(grid,),
        in_specs=[
            pl.BlockSpec((1, 1, b), lambda i: (i, 0, 0)),
            pl.BlockSpec((1, 1, b), lambda i: (i, 0, 0)),
            pl.BlockSpec((r, b), lambda i: (0, i)),
            pl.BlockSpec((c, k), lambda i: (0, 0)),
        ],
        out_specs=pl.BlockSpec((c, b), lambda i: (0, i)),
        out_shape=jax.ShapeDtypeStruct((c, n_edges), jnp.float32),
    )(zs_v, zd_v, e_t, wtilde_t)
    return out_t.T
